# baseline (speedup 1.0000x reference)
import functools

import jax
import jax.numpy as jnp
from jax import lax
from jax.experimental import pallas as pl
from jax.experimental.pallas import tpu as pltpu
from jax.experimental.pallas import tpu_sc as plsc

BATCH = 4
SEQ = 8192
DIM = 1024
TOT = BATCH * SEQ

_info = plsc.get_sparse_core_info()
NC, NS = _info.num_cores, _info.num_subcores
NW = NC * NS
PER_W = TOT // NW
CHUNK = 16
NCHUNK = PER_W // CHUNK
NBUF = 2

_mesh = plsc.VectorSubcoreMesh(core_axis_name="c", subcore_axis_name="s")


@functools.partial(
    pl.kernel,
    mesh=_mesh,
    out_type=jax.ShapeDtypeStruct((TOT, DIM), jnp.float32),
    scratch_types=[
        pltpu.VMEM((NCHUNK, CHUNK), jnp.int32),
    ] + [pltpu.VMEM((CHUNK, DIM), jnp.float32)] * NBUF
      + [pltpu.SemaphoreType.DMA] * (2 * NBUF),
)
def _gather_rows(idx_hbm, table_hbm, out_hbm, idx_v, *bufs_and_sems):
    bufs = bufs_and_sems[:NBUF]
    gsems = bufs_and_sems[NBUF:2 * NBUF]
    ssems = bufs_and_sems[2 * NBUF:]
    wid = lax.axis_index("s") * NC + lax.axis_index("c")
    base = wid * PER_W
    pltpu.sync_copy(idx_hbm.at[wid], idx_v)

    def g_start(c, b):
        pltpu.async_copy(table_hbm.at[idx_v.at[c]], bufs[b], gsems[b])

    def g_wait(c, b):
        pltpu.make_async_copy(table_hbm.at[idx_v.at[c]], bufs[b],
                              gsems[b]).wait()

    def out_slice(c):
        return out_hbm.at[pl.ds(base + c * CHUNK, CHUNK)]

    def s_start(c, b):
        pltpu.async_copy(bufs[b], out_slice(c), ssems[b])

    def s_wait(c, b):
        pltpu.make_async_copy(bufs[b], out_slice(c), ssems[b]).wait()

    for b in range(NBUF):
        g_start(b, b)

    def body(i, carry):
        cc = i * NBUF
        for b in range(NBUF):
            g_wait(cc + b, b)
            s_start(cc + b, b)
        for b in range(NBUF):
            s_wait(cc + b, b)
            g_start(cc + b + NBUF, b)
        return carry

    lax.fori_loop(0, (NCHUNK - NBUF) // NBUF, body, 0)

    last = NCHUNK - NBUF
    for b in range(NBUF):
        g_wait(last + b, b)
        s_start(last + b, b)
    for b in range(NBUF):
        s_wait(last + b, b)


def kernel(indices, table):
    idx = indices.astype(jnp.int32).reshape(NW, NCHUNK, CHUNK)
    out = _gather_rows(idx, table)
    return out.reshape(BATCH, SEQ, DIM)


# speedup vs baseline: 1.0713x; 1.0713x over previous
import functools

import jax
import jax.numpy as jnp
from jax import lax
from jax.experimental import pallas as pl
from jax.experimental.pallas import tpu as pltpu
from jax.experimental.pallas import tpu_sc as plsc

BATCH = 4
SEQ = 8192
DIM = 1024
TOT = BATCH * SEQ

_info = plsc.get_sparse_core_info()
NC, NS = _info.num_cores, _info.num_subcores
NW = NC * NS
PER_W = TOT // NW
CHUNK = 8
NCHUNK = PER_W // CHUNK
NBUF = 8

_mesh = plsc.VectorSubcoreMesh(core_axis_name="c", subcore_axis_name="s")


@functools.partial(
    pl.kernel,
    mesh=_mesh,
    out_type=jax.ShapeDtypeStruct((TOT, DIM), jnp.float32),
    scratch_types=[
        pltpu.VMEM((NCHUNK, CHUNK), jnp.int32),
    ] + [pltpu.VMEM((CHUNK, DIM), jnp.float32)] * NBUF
      + [pltpu.SemaphoreType.DMA] * (2 * NBUF),
)
def _gather_rows(idx_hbm, table_hbm, out_hbm, idx_v, *bufs_and_sems):
    bufs = bufs_and_sems[:NBUF]
    gsems = bufs_and_sems[NBUF:2 * NBUF]
    ssems = bufs_and_sems[2 * NBUF:]
    wid = lax.axis_index("s") * NC + lax.axis_index("c")
    base = wid * PER_W
    pltpu.sync_copy(idx_hbm.at[wid], idx_v)

    def g_start(c, b):
        pltpu.async_copy(table_hbm.at[idx_v.at[c]], bufs[b], gsems[b])

    def g_wait(c, b):
        pltpu.make_async_copy(table_hbm.at[idx_v.at[c]], bufs[b],
                              gsems[b]).wait()

    def out_slice(c):
        return out_hbm.at[pl.ds(base + c * CHUNK, CHUNK)]

    def s_start(c, b):
        pltpu.async_copy(bufs[b], out_slice(c), ssems[b])

    def s_wait(c, b):
        pltpu.make_async_copy(bufs[b], out_slice(c), ssems[b]).wait()

    for b in range(NBUF):
        g_start(b, b)

    def body(i, carry):
        cc = i * NBUF
        for b in range(NBUF):
            g_wait(cc + b, b)
            s_start(cc + b, b)
        for b in range(NBUF):
            s_wait(cc + b, b)
            g_start(cc + b + NBUF, b)
        return carry

    lax.fori_loop(0, (NCHUNK - NBUF) // NBUF, body, 0)

    last = NCHUNK - NBUF
    for b in range(NBUF):
        g_wait(last + b, b)
        s_start(last + b, b)
    for b in range(NBUF):
        s_wait(last + b, b)


def kernel(indices, table):
    idx = indices.astype(jnp.int32).reshape(NW, NCHUNK, CHUNK)
    out = _gather_rows(idx, table)
    return out.reshape(BATCH, SEQ, DIM)
